# baseline (speedup 1.0000x reference)
import functools

import jax
import jax.numpy as jnp
from jax import lax
from jax.experimental import pallas as pl
from jax.experimental.pallas import tpu as pltpu
from jax.experimental.pallas import tpu_sc as plsc

N = 10000
NP = 10240
DUMP = N
EP = 172032
B = 64
SCI = 12
ITERS1 = 168
ITERS2 = 84
NCH1 = ITERS1 // SCI
NCH2 = ITERS2 // SCI
ROWS_PT = NP // 16
NPDU = NP // 32
NPD = 512
DROWS_PT = NPD // 16
BLK = 1024

_i32 = jnp.int32
_f32 = jnp.float32


def _iota16():
    return lax.iota(_i32, 16)


def _zeros16():
    return jnp.zeros((16,), _f32)


def _full16(v):
    return jnp.full((16,), v, _i32)


def _proj_body(x_ref, w_ref, xl_ref, xr_ref):
    h = jnp.dot(x_ref[...], w_ref[...], preferred_element_type=_f32)
    blk = x_ref.shape[0]
    xl_ref[...] = h[:, :256].reshape(2 * blk, 128)
    xr_ref[...] = h[:, 256:].reshape(2 * blk, 128)


def _proj(x_pad, wcat):
    return pl.pallas_call(
        _proj_body,
        grid=(NP // BLK,),
        in_specs=[
            pl.BlockSpec((BLK, 256), lambda i: (i, 0)),
            pl.BlockSpec((256, 512), lambda i: (0, 0)),
        ],
        out_specs=[
            pl.BlockSpec((2 * BLK, 128), lambda i: (i, 0)),
            pl.BlockSpec((2 * BLK, 128), lambda i: (i, 0)),
        ],
        out_shape=[
            jax.ShapeDtypeStruct((2 * NP, 128), _f32),
            jax.ShapeDtypeStruct((2 * NP, 128), _f32),
        ],
    )(x_pad, wcat)


def _mid_body(num_ref, den_ref, b1_ref, w_ref, xl2_ref, xr2_ref):
    num = num_ref[...]
    den = den_ref[...]
    hh = lax.broadcasted_iota(_i32, (4, 128), 0)
    cc = lax.broadcasted_iota(_i32, (4, 128), 1) // 32
    rmat = jnp.where(hh == cc, 1.0, 0.0).astype(_f32)
    h0 = num[0] / (jnp.dot(den[0], rmat, preferred_element_type=_f32) + 1e-16)
    h1 = num[1] / (jnp.dot(den[1], rmat, preferred_element_type=_f32) + 1e-16)
    h = jnp.concatenate([h0, h1], axis=-1) + b1_ref[...]
    h = jnp.where(h > 0, h, 0.01 * h)
    z = jnp.dot(h, w_ref[...], preferred_element_type=_f32)
    zz = jnp.zeros_like(z[:, :64])
    xl2_ref[...] = jnp.concatenate([z[:, :64], zz], axis=-1)
    xr2_ref[...] = jnp.concatenate([z[:, 64:], zz], axis=-1)


def _mid(num1, den1, b1, wcat2):
    return pl.pallas_call(
        _mid_body,
        grid=(NP // BLK,),
        in_specs=[
            pl.BlockSpec((2, BLK, 128), lambda i: (0, i, 0)),
            pl.BlockSpec((2, BLK, 4), lambda i: (0, i, 0)),
            pl.BlockSpec((1, 256), lambda i: (0, 0)),
            pl.BlockSpec((256, 128), lambda i: (0, 0)),
        ],
        out_specs=[
            pl.BlockSpec((BLK, 128), lambda i: (i, 0)),
            pl.BlockSpec((BLK, 128), lambda i: (i, 0)),
        ],
        out_shape=[
            jax.ShapeDtypeStruct((NP, 128), _f32),
            jax.ShapeDtypeStruct((NP, 128), _f32),
        ],
    )(num1, den1, b1, wcat2)


def _head_body(num_ref, den_ref, b2_ref, wlin_ref, blin_ref, out_ref, prob_ref):
    num = num_ref[...]
    den = den_ref[...]
    d = den[0, :, 0:1] + den[1, :, 0:1]
    h2 = (num[0, :, :64] + num[1, :, :64]) / (d + 1e-16) + b2_ref[...]
    h2 = jnp.maximum(h2, 0.0)
    z = jnp.dot(h2, wlin_ref[...], preferred_element_type=_f32) + blin_ref[...]
    out_ref[...] = z
    m = jnp.max(z, axis=-1, keepdims=True)
    ez = jnp.exp(z - m)
    prob_ref[...] = ez / jnp.sum(ez, axis=-1, keepdims=True)


def _head(num2, den2, b2, wlin, blin):
    return pl.pallas_call(
        _head_body,
        grid=(NP // BLK,),
        in_specs=[
            pl.BlockSpec((2, BLK, 128), lambda i: (0, i, 0)),
            pl.BlockSpec((2, BLK, 4), lambda i: (0, i, 0)),
            pl.BlockSpec((1, 64), lambda i: (0, 0)),
            pl.BlockSpec((64, 16), lambda i: (0, 0)),
            pl.BlockSpec((1, 16), lambda i: (0, 0)),
        ],
        out_specs=[
            pl.BlockSpec((BLK, 16), lambda i: (i, 0)),
            pl.BlockSpec((BLK, 16), lambda i: (i, 0)),
        ],
        out_shape=[
            jax.ShapeDtypeStruct((NP, 16), _f32),
            jax.ShapeDtypeStruct((NP, 16), _f32),
        ],
    )(num2, den2, b2, wlin, blin)


def _zero_rows(buf, nrows, cols):
    def rzody(r, carry):
        for j in range(cols // 16):
            plsc.store_scatter(buf, [_full16(0) + r, j * 16 + _iota16()],
                               _zeros16())
        return carry
    lax.fori_loop(0, nrows, rzody, 0)


def _zero_acc(rows_l, acc_num, acc_den, row0, drow0):
    def zbody(k, carry):
        pltpu.sync_copy(rows_l, acc_num.at[pl.ds(row0 + k * B, B)])
        return carry
    lax.fori_loop(0, ROWS_PT // B, zbody, 0)
    pltpu.sync_copy(rows_l.at[pl.ds(0, DROWS_PT)],
                    acc_den.at[pl.ds(drow0, DROWS_PT)])


def _edge_l1(xl_hbm, xr_hbm, src_hbm, dst_hbm, att_hbm,
             num_hbm, den_hbm,
             acc_num, acc_den,
             src1d, dst1d,
             il0, il1, ir0, ir1, dd0, dd1, dv0, dv1,
             rl0, rl1, rr0, rr1, den_v,
             ex_v, att_v, sl0, sl1, sr0, sr1):
    c = lax.axis_index("c")
    s = lax.axis_index("s")
    row0 = s * ROWS_PT
    drow0 = s * DROWS_PT
    ilb = (il0, il1)
    irb = (ir0, ir1)
    ddb = (dd0, dd1)
    dvb = (dv0, dv1)
    rlb = (rl0, rl1)
    rrb = (rr0, rr1)
    slb = (sl0, sl1)
    srb = (sr0, sr1)

    _zero_rows(den_v, B, 128)
    _zero_rows(rl0, B, 128)
    _zero_acc(rl0, acc_num, acc_den, row0, drow0)
    pltpu.sync_copy(att_hbm.at[pl.ds(c * 128, 128)], att_v)
    plsc.subcore_barrier()

    def make_idx(j, bi):
        for k in range(B // 16):
            lane = k * 16 + _iota16()
            sv = plsc.load_gather(src1d, [j * B + lane])
            plsc.store_scatter(ilb[bi], [lane], sv * 2 + c)
            dv = plsc.load_gather(dst1d, [j * B + lane])
            plsc.store_scatter(irb[bi], [lane], dv * 2 + c)
            plsc.store_scatter(ddb[bi], [lane], dv // 32)
            plsc.store_scatter(dvb[bi], [lane], dv)

    def issue(bi):
        pltpu.async_copy(xl_hbm.at[ilb[bi]], rlb[bi], slb[bi])
        pltpu.async_copy(xr_hbm.at[irb[bi]], rrb[bi], srb[bi])

    def wait(bi):
        pltpu.make_async_copy(xl_hbm.at[ilb[bi]], rlb[bi], slb[bi]).wait()
        pltpu.make_async_copy(xr_hbm.at[irb[bi]], rrb[bi], srb[bi]).wait()

    def compute(bi):
        rows_l = rlb[bi]
        rows_r = rrb[bi]
        for g in range(B // 16):
            rows16 = _full16(g * 16) + _iota16()
            dstg = dvb[bi][pl.ds(g * 16, 16)]
            colb = (dstg - (dstg // 32) * 32) * 4
            for h in range(4):
                def abody(t, a):
                    cb = h * 32 + t * 8
                    for u in range(8):
                        cid = _full16(0) + (cb + u)
                        ml = plsc.load_gather(rows_l, [rows16, cid])
                        mr = plsc.load_gather(rows_r, [rows16, cid])
                        m = ml + mr
                        m = jnp.where(m > 0, m, m * 0.2)
                        a = a + plsc.load_gather(att_v, [cid]) * m
                    return a
                a = lax.fori_loop(0, 4, abody, _zeros16())
                ex = jnp.exp(a)
                plsc.store_scatter(den_v, [rows16, colb + h], ex)

                def sbody(t, carry2):
                    cb = h * 32 + t * 8
                    for u in range(8):
                        cid = _full16(0) + (cb + u)
                        rl = plsc.load_gather(rows_l, [rows16, cid])
                        plsc.store_scatter(rows_l, [rows16, cid], ex * rl)
                    return carry2
                lax.fori_loop(0, 4, sbody, 0)
        pltpu.sync_copy(rows_l, acc_num.at[dvb[bi]], add=True)
        pltpu.sync_copy(den_v, acc_den.at[ddb[bi]], add=True)
        for g in range(B // 16):
            rows16 = _full16(g * 16) + _iota16()
            dstg = dvb[bi][pl.ds(g * 16, 16)]
            colb = (dstg - (dstg // 32) * 32) * 4
            for h in range(4):
                plsc.store_scatter(den_v, [rows16, colb + h], _zeros16())

    def chunk(p, carry):
        ebase = s * (ITERS1 * B) + p * (SCI * B)
        pltpu.sync_copy(src_hbm.at[pl.ds(ebase, SCI * B)], src1d)
        pltpu.sync_copy(dst_hbm.at[pl.ds(ebase, SCI * B)], dst1d)
        make_idx(0, 0)
        issue(0)

        def jpair(q, carry2):
            j0 = q * 2
            wait(0)
            make_idx(j0 + 1, 1)
            issue(1)
            compute(0)
            wait(1)

            @pl.when(q < SCI // 2 - 1)
            def _prefetch():
                make_idx(j0 + 2, 0)
                issue(0)
            compute(1)
            return carry2
        lax.fori_loop(0, SCI // 2, jpair, 0)
        return carry
    lax.fori_loop(0, NCH1, chunk, 0)

    plsc.subcore_barrier()
    pltpu.sync_copy(acc_num.at[pl.ds(row0, ROWS_PT)],
                    num_hbm.at[pl.ds(c * NP + row0, ROWS_PT)])
    pltpu.sync_copy(acc_den.at[pl.ds(drow0, DROWS_PT)],
                    den_hbm.at[pl.ds(c * NPD + drow0, DROWS_PT)])


def _edge_l2(xl_hbm, xr_hbm, src_hbm, dst_hbm, att_hbm,
             num_hbm, den_hbm,
             acc_num, acc_den,
             src1d, dst1d,
             il0, il1, ir0, ir1, dd0, dd1,
             rl0, rl1, rr0, rr1, den_v,
             ex_v, att_v, sl0, sl1, sr0, sr1):
    c = lax.axis_index("c")
    s = lax.axis_index("s")
    row0 = s * ROWS_PT
    drow0 = s * DROWS_PT
    ilb = (il0, il1)
    irb = (ir0, ir1)
    ddb = (dd0, dd1)
    rlb = (rl0, rl1)
    rrb = (rr0, rr1)
    slb = (sl0, sl1)
    srb = (sr0, sr1)

    _zero_rows(den_v, B, 128)
    _zero_rows(rl0, B, 128)
    _zero_acc(rl0, acc_num, acc_den, row0, drow0)
    pltpu.sync_copy(att_hbm, att_v)
    plsc.subcore_barrier()

    def make_idx(j, bi):
        for k in range(B // 16):
            lane = k * 16 + _iota16()
            sv = plsc.load_gather(src1d, [j * B + lane])
            plsc.store_scatter(ilb[bi], [lane], sv)
            dv = plsc.load_gather(dst1d, [j * B + lane])
            plsc.store_scatter(irb[bi], [lane], dv)
            plsc.store_scatter(ddb[bi], [lane], dv // 32)

    def issue(bi):
        pltpu.async_copy(xl_hbm.at[ilb[bi]], rlb[bi], slb[bi])
        pltpu.async_copy(xr_hbm.at[irb[bi]], rrb[bi], srb[bi])

    def wait(bi):
        pltpu.make_async_copy(xl_hbm.at[ilb[bi]], rlb[bi], slb[bi]).wait()
        pltpu.make_async_copy(xr_hbm.at[irb[bi]], rrb[bi], srb[bi]).wait()

    def compute(bi):
        rows_l = rlb[bi]
        rows_r = rrb[bi]
        for g in range(B // 16):
            rows16 = _full16(g * 16) + _iota16()
            dstg = irb[bi][pl.ds(g * 16, 16)]
            colb = (dstg - (dstg // 32) * 32) * 4

            def abody(t, a):
                cb = t * 8
                for u in range(8):
                    cid = _full16(0) + (cb + u)
                    ml = plsc.load_gather(rows_l, [rows16, cid])
                    mr = plsc.load_gather(rows_r, [rows16, cid])
                    m = ml + mr
                    m = jnp.where(m > 0, m, m * 0.2)
                    a = a + plsc.load_gather(att_v, [cid]) * m
                return a
            a = lax.fori_loop(0, 8, abody, _zeros16())
            ex = jnp.exp(a)
            plsc.store_scatter(den_v, [rows16, colb], ex)

            def sbody(t, carry2):
                cb = t * 8
                for u in range(8):
                    cid = _full16(0) + (cb + u)
                    rl = plsc.load_gather(rows_l, [rows16, cid])
                    plsc.store_scatter(rows_l, [rows16, cid], ex * rl)
                return carry2
            lax.fori_loop(0, 8, sbody, 0)
        pltpu.sync_copy(rows_l, acc_num.at[irb[bi]], add=True)
        pltpu.sync_copy(den_v, acc_den.at[ddb[bi]], add=True)
        for g in range(B // 16):
            rows16 = _full16(g * 16) + _iota16()
            dstg = irb[bi][pl.ds(g * 16, 16)]
            colb = (dstg - (dstg // 32) * 32) * 4
            plsc.store_scatter(den_v, [rows16, colb], _zeros16())

    def chunk(p, carry):
        ebase = c * (EP // 2) + s * (ITERS2 * B) + p * (SCI * B)
        pltpu.sync_copy(src_hbm.at[pl.ds(ebase, SCI * B)], src1d)
        pltpu.sync_copy(dst_hbm.at[pl.ds(ebase, SCI * B)], dst1d)
        make_idx(0, 0)
        issue(0)

        def jpair(q, carry2):
            j0 = q * 2
            wait(0)
            make_idx(j0 + 1, 1)
            issue(1)
            compute(0)
            wait(1)

            @pl.when(q < SCI // 2 - 1)
            def _prefetch():
                make_idx(j0 + 2, 0)
                issue(0)
            compute(1)
            return carry2
        lax.fori_loop(0, SCI // 2, jpair, 0)
        return carry
    lax.fori_loop(0, NCH2, chunk, 0)

    plsc.subcore_barrier()
    pltpu.sync_copy(acc_num.at[pl.ds(row0, ROWS_PT)],
                    num_hbm.at[pl.ds(c * NP + row0, ROWS_PT)])
    pltpu.sync_copy(acc_den.at[pl.ds(drow0, DROWS_PT)],
                    den_hbm.at[pl.ds(c * NPD + drow0, DROWS_PT)])


def _sc_mesh():
    return plsc.VectorSubcoreMesh(core_axis_name="c", subcore_axis_name="s")


def _edge_phase1(xl_i, xr_i, src_e, dst_e, att1f):
    f = pl.kernel(
        _edge_l1,
        out_type=[
            jax.ShapeDtypeStruct((2 * NP, 128), _f32),
            jax.ShapeDtypeStruct((2 * NPD, 128), _f32),
        ],
        mesh=_sc_mesh(),
        scratch_types=[
            pltpu.VMEM_SHARED((NP, 128), _f32),
            pltpu.VMEM_SHARED((NPD, 128), _f32),
            pltpu.VMEM((SCI * B,), _i32),
            pltpu.VMEM((SCI * B,), _i32),
            pltpu.VMEM((B,), _i32),
            pltpu.VMEM((B,), _i32),
            pltpu.VMEM((B,), _i32),
            pltpu.VMEM((B,), _i32),
            pltpu.VMEM((B,), _i32),
            pltpu.VMEM((B,), _i32),
            pltpu.VMEM((B,), _i32),
            pltpu.VMEM((B,), _i32),
            pltpu.VMEM((B, 128), _f32),
            pltpu.VMEM((B, 128), _f32),
            pltpu.VMEM((B, 128), _f32),
            pltpu.VMEM((B, 128), _f32),
            pltpu.VMEM((B, 128), _f32),
            pltpu.VMEM((64,), _f32),
            pltpu.VMEM((128,), _f32),
            pltpu.SemaphoreType.DMA,
            pltpu.SemaphoreType.DMA,
            pltpu.SemaphoreType.DMA,
            pltpu.SemaphoreType.DMA,
        ],
        compiler_params=pltpu.CompilerParams(needs_layout_passes=False),
    )
    return f(xl_i, xr_i, src_e, dst_e, att1f)


def _edge_phase2(xl2, xr2, src_e, dst_e, att2f):
    f = pl.kernel(
        _edge_l2,
        out_type=[
            jax.ShapeDtypeStruct((2 * NP, 128), _f32),
            jax.ShapeDtypeStruct((2 * NPD, 128), _f32),
        ],
        mesh=_sc_mesh(),
        scratch_types=[
            pltpu.VMEM_SHARED((NP, 128), _f32),
            pltpu.VMEM_SHARED((NPD, 128), _f32),
            pltpu.VMEM((SCI * B,), _i32),
            pltpu.VMEM((SCI * B,), _i32),
            pltpu.VMEM((B,), _i32),
            pltpu.VMEM((B,), _i32),
            pltpu.VMEM((B,), _i32),
            pltpu.VMEM((B,), _i32),
            pltpu.VMEM((B,), _i32),
            pltpu.VMEM((B,), _i32),
            pltpu.VMEM((B, 128), _f32),
            pltpu.VMEM((B, 128), _f32),
            pltpu.VMEM((B, 128), _f32),
            pltpu.VMEM((B, 128), _f32),
            pltpu.VMEM((B, 128), _f32),
            pltpu.VMEM((16,), _f32),
            pltpu.VMEM((64,), _f32),
            pltpu.SemaphoreType.DMA,
            pltpu.SemaphoreType.DMA,
            pltpu.SemaphoreType.DMA,
            pltpu.SemaphoreType.DMA,
        ],
        compiler_params=pltpu.CompilerParams(needs_layout_passes=False),
    )
    return f(xl2, xr2, src_e, dst_e, att2f)


def kernel(x, edge_index, Wl1, Wr1, att1, b1, Wl2, Wr2, att2, b2, Wlin, blin):
    x_pad = jnp.zeros((NP, 256), _f32).at[:N].set(x.astype(_f32))
    ei = edge_index.astype(_i32)
    self_i = jnp.arange(N, dtype=_i32)
    e_raw = ei.shape[1]
    pad = jnp.full((EP - e_raw - N,), DUMP, _i32)
    src_e = jnp.concatenate([ei[0], self_i, pad])
    dst_e = jnp.concatenate([ei[1], self_i, pad])

    wcat1 = jnp.concatenate([Wl1, Wr1], axis=1)
    xl_i, xr_i = _proj(x_pad, wcat1)

    att1f = att1.reshape(256).astype(_f32)
    num1, den1 = _edge_phase1(xl_i, xr_i, src_e, dst_e, att1f)
    num1 = num1.reshape(2, NP, 128)
    den1 = den1.reshape(2, NPD, 128)[:, :NPDU]
    den1 = den1.reshape(2, NP, 4)

    wcat2 = jnp.concatenate([Wl2, Wr2], axis=1)
    xl2, xr2 = _mid(num1, den1, b1.reshape(1, 256), wcat2)

    att2f = att2.reshape(64).astype(_f32)
    num2, den2 = _edge_phase2(xl2, xr2, src_e, dst_e, att2f)
    num2 = num2.reshape(2, NP, 128)
    den2 = den2.reshape(2, NPD, 128)[:, :NPDU]
    den2 = den2.reshape(2, NP, 4)

    out, prob = _head(num2, den2, b2.reshape(1, 64), Wlin,
                      blin.reshape(1, 16))
    return (out[:N], prob[:N])


# speedup vs baseline: 1.5398x; 1.5398x over previous
import functools

import jax
import jax.numpy as jnp
from jax import lax
from jax.experimental import pallas as pl
from jax.experimental.pallas import tpu as pltpu
from jax.experimental.pallas import tpu_sc as plsc

N = 10000
NP = 10240
DUMP = N
EP = 172032
B = 64
SCI = 12
ITERS1 = 168
ITERS2 = 84
NCH1 = ITERS1 // SCI
NCH2 = ITERS2 // SCI
ROWS_PT = NP // 16
NPDU = NP // 32
NPD = 512
DROWS_PT = NPD // 16
BLK = 1024

_i32 = jnp.int32
_f32 = jnp.float32


def _iota16():
    return lax.iota(_i32, 16)


def _zeros16():
    return jnp.zeros((16,), _f32)


def _full16(v):
    return jnp.full((16,), v, _i32)


def _proj_body(x_ref, w_ref, xl_ref, xr_ref):
    h = jnp.dot(x_ref[...], w_ref[...], preferred_element_type=_f32)
    blk = x_ref.shape[0]
    xl_ref[...] = h[:, :256].reshape(2 * blk, 128)
    xr_ref[...] = h[:, 256:].reshape(2 * blk, 128)


def _proj(x_pad, wcat):
    return pl.pallas_call(
        _proj_body,
        grid=(NP // BLK,),
        in_specs=[
            pl.BlockSpec((BLK, 256), lambda i: (i, 0)),
            pl.BlockSpec((256, 512), lambda i: (0, 0)),
        ],
        out_specs=[
            pl.BlockSpec((2 * BLK, 128), lambda i: (i, 0)),
            pl.BlockSpec((2 * BLK, 128), lambda i: (i, 0)),
        ],
        out_shape=[
            jax.ShapeDtypeStruct((2 * NP, 128), _f32),
            jax.ShapeDtypeStruct((2 * NP, 128), _f32),
        ],
    )(x_pad, wcat)


def _mid_body(num_ref, den_ref, b1_ref, w_ref, xl2_ref, xr2_ref):
    num = num_ref[...]
    den = den_ref[...]
    hh = lax.broadcasted_iota(_i32, (4, 128), 0)
    cc = lax.broadcasted_iota(_i32, (4, 128), 1) // 32
    rmat = jnp.where(hh == cc, 1.0, 0.0).astype(_f32)
    h0 = num[0] / (jnp.dot(den[0], rmat, preferred_element_type=_f32) + 1e-16)
    h1 = num[1] / (jnp.dot(den[1], rmat, preferred_element_type=_f32) + 1e-16)
    h = jnp.concatenate([h0, h1], axis=-1) + b1_ref[...]
    h = jnp.where(h > 0, h, 0.01 * h)
    z = jnp.dot(h, w_ref[...], preferred_element_type=_f32)
    zz = jnp.zeros_like(z[:, :64])
    xl2_ref[...] = jnp.concatenate([z[:, :64], zz], axis=-1)
    xr2_ref[...] = jnp.concatenate([z[:, 64:], zz], axis=-1)


def _mid(num1, den1, b1, wcat2):
    return pl.pallas_call(
        _mid_body,
        grid=(NP // BLK,),
        in_specs=[
            pl.BlockSpec((2, BLK, 128), lambda i: (0, i, 0)),
            pl.BlockSpec((2, BLK, 4), lambda i: (0, i, 0)),
            pl.BlockSpec((1, 256), lambda i: (0, 0)),
            pl.BlockSpec((256, 128), lambda i: (0, 0)),
        ],
        out_specs=[
            pl.BlockSpec((BLK, 128), lambda i: (i, 0)),
            pl.BlockSpec((BLK, 128), lambda i: (i, 0)),
        ],
        out_shape=[
            jax.ShapeDtypeStruct((NP, 128), _f32),
            jax.ShapeDtypeStruct((NP, 128), _f32),
        ],
    )(num1, den1, b1, wcat2)


def _head_body(num_ref, den_ref, b2_ref, wlin_ref, blin_ref, out_ref, prob_ref):
    num = num_ref[...]
    den = den_ref[...]
    d = den[0, :, 0:1] + den[1, :, 0:1]
    h2 = (num[0, :, :64] + num[1, :, :64]) / (d + 1e-16) + b2_ref[...]
    h2 = jnp.maximum(h2, 0.0)
    z = jnp.dot(h2, wlin_ref[...], preferred_element_type=_f32) + blin_ref[...]
    out_ref[...] = z
    m = jnp.max(z, axis=-1, keepdims=True)
    ez = jnp.exp(z - m)
    prob_ref[...] = ez / jnp.sum(ez, axis=-1, keepdims=True)


def _head(num2, den2, b2, wlin, blin):
    return pl.pallas_call(
        _head_body,
        grid=(NP // BLK,),
        in_specs=[
            pl.BlockSpec((2, BLK, 128), lambda i: (0, i, 0)),
            pl.BlockSpec((2, BLK, 4), lambda i: (0, i, 0)),
            pl.BlockSpec((1, 64), lambda i: (0, 0)),
            pl.BlockSpec((64, 16), lambda i: (0, 0)),
            pl.BlockSpec((1, 16), lambda i: (0, 0)),
        ],
        out_specs=[
            pl.BlockSpec((BLK, 16), lambda i: (i, 0)),
            pl.BlockSpec((BLK, 16), lambda i: (i, 0)),
        ],
        out_shape=[
            jax.ShapeDtypeStruct((NP, 16), _f32),
            jax.ShapeDtypeStruct((NP, 16), _f32),
        ],
    )(num2, den2, b2, wlin, blin)


def _zero_rows(buf, nrows, cols):
    def rzody(r, carry):
        for j in range(cols // 16):
            plsc.store_scatter(buf, [_full16(0) + r, j * 16 + _iota16()],
                               _zeros16())
        return carry
    lax.fori_loop(0, nrows, rzody, 0)


def _zero_acc(rows_l, acc_num, acc_den, row0, drow0):
    def zbody(k, carry):
        pltpu.sync_copy(rows_l, acc_num.at[pl.ds(row0 + k * B, B)])
        return carry
    lax.fori_loop(0, ROWS_PT // B, zbody, 0)
    pltpu.sync_copy(rows_l.at[pl.ds(0, DROWS_PT)],
                    acc_den.at[pl.ds(drow0, DROWS_PT)])


def _edge_l1(xl_hbm, xr_hbm, src_hbm, dst_hbm, att_hbm,
             num_hbm, den_hbm,
             acc_num, acc_den,
             src1d, dst1d,
             il0, il1, ir0, ir1, dd0, dd1, dv0, dv1,
             rl0, rl1, rr0, rr1, den_v,
             ex_v, att_v, sl0, sl1, sr0, sr1):
    c = lax.axis_index("c")
    s = lax.axis_index("s")
    row0 = s * ROWS_PT
    drow0 = s * DROWS_PT
    ilb = (il0, il1)
    irb = (ir0, ir1)
    ddb = (dd0, dd1)
    dvb = (dv0, dv1)
    rlb = (rl0, rl1)
    rrb = (rr0, rr1)
    slb = (sl0, sl1)
    srb = (sr0, sr1)

    _zero_rows(den_v, B, 128)
    _zero_rows(rl0, B, 128)
    _zero_acc(rl0, acc_num, acc_den, row0, drow0)
    pltpu.sync_copy(att_hbm.at[pl.ds(c * 128, 128)], att_v)
    plsc.subcore_barrier()

    def make_idx(j, bi):
        for k in range(B // 16):
            lane = k * 16 + _iota16()
            sv = plsc.load_gather(src1d, [j * B + lane])
            plsc.store_scatter(ilb[bi], [lane], sv * 2 + c)
            dv = plsc.load_gather(dst1d, [j * B + lane])
            plsc.store_scatter(irb[bi], [lane], dv * 2 + c)
            plsc.store_scatter(ddb[bi], [lane], dv // 32)
            plsc.store_scatter(dvb[bi], [lane], dv)

    def issue(bi):
        pltpu.async_copy(xl_hbm.at[ilb[bi]], rlb[bi], slb[bi])
        pltpu.async_copy(xr_hbm.at[irb[bi]], rrb[bi], srb[bi])

    def wait(bi):
        pltpu.make_async_copy(xl_hbm.at[ilb[bi]], rlb[bi], slb[bi]).wait()
        pltpu.make_async_copy(xr_hbm.at[irb[bi]], rrb[bi], srb[bi]).wait()

    def compute(bi):
        rows_l = rlb[bi]
        rows_r = rrb[bi]
        for g in range(B // 16):
            rows16 = _full16(g * 16) + _iota16()
            dstg = dvb[bi][pl.ds(g * 16, 16)]
            colb = (dstg - (dstg // 32) * 32) * 4
            for h in range(4):
                def abody(t, a):
                    cb = h * 32 + t * 8
                    for u in range(8):
                        cid = _full16(0) + (cb + u)
                        ml = plsc.load_gather(rows_l, [rows16, cid])
                        mr = plsc.load_gather(rows_r, [rows16, cid])
                        m = ml + mr
                        m = jnp.where(m > 0, m, m * 0.2)
                        a = a + plsc.load_gather(att_v, [cid]) * m
                    return a
                a = lax.fori_loop(0, 4, abody, _zeros16())
                ex = jnp.exp(a)
                ex_v[pl.ds(h * 16, 16)] = ex
                plsc.store_scatter(den_v, [rows16, colb + h], ex)

            def mbody(ei, carry2):
                rowv = _full16(g * 16) + ei
                for h in range(4):
                    exb = plsc.load_gather(ex_v, [_full16(h * 16) + ei])
                    for jj in range(2):
                        cols = _full16(h * 32 + jj * 16) + _iota16()
                        rl = plsc.load_gather(rows_l, [rowv, cols])
                        plsc.store_scatter(rows_l, [rowv, cols], exb * rl)
                return carry2
            lax.fori_loop(0, 16, mbody, 0)
        pltpu.sync_copy(rows_l, acc_num.at[dvb[bi]], add=True)
        pltpu.sync_copy(den_v, acc_den.at[ddb[bi]], add=True)
        for g in range(B // 16):
            rows16 = _full16(g * 16) + _iota16()
            dstg = dvb[bi][pl.ds(g * 16, 16)]
            colb = (dstg - (dstg // 32) * 32) * 4
            for h in range(4):
                plsc.store_scatter(den_v, [rows16, colb + h], _zeros16())

    def chunk(p, carry):
        ebase = s * (ITERS1 * B) + p * (SCI * B)
        pltpu.sync_copy(src_hbm.at[pl.ds(ebase, SCI * B)], src1d)
        pltpu.sync_copy(dst_hbm.at[pl.ds(ebase, SCI * B)], dst1d)
        make_idx(0, 0)
        issue(0)

        def jpair(q, carry2):
            j0 = q * 2
            wait(0)
            make_idx(j0 + 1, 1)
            issue(1)
            compute(0)
            wait(1)

            @pl.when(q < SCI // 2 - 1)
            def _prefetch():
                make_idx(j0 + 2, 0)
                issue(0)
            compute(1)
            return carry2
        lax.fori_loop(0, SCI // 2, jpair, 0)
        return carry
    lax.fori_loop(0, NCH1, chunk, 0)

    plsc.subcore_barrier()
    pltpu.sync_copy(acc_num.at[pl.ds(row0, ROWS_PT)],
                    num_hbm.at[pl.ds(c * NP + row0, ROWS_PT)])
    pltpu.sync_copy(acc_den.at[pl.ds(drow0, DROWS_PT)],
                    den_hbm.at[pl.ds(c * NPD + drow0, DROWS_PT)])


def _edge_l2(xl_hbm, xr_hbm, src_hbm, dst_hbm, att_hbm,
             num_hbm, den_hbm,
             acc_num, acc_den,
             src1d, dst1d,
             il0, il1, ir0, ir1, dd0, dd1,
             rl0, rl1, rr0, rr1, den_v,
             ex_v, att_v, sl0, sl1, sr0, sr1):
    c = lax.axis_index("c")
    s = lax.axis_index("s")
    row0 = s * ROWS_PT
    drow0 = s * DROWS_PT
    ilb = (il0, il1)
    irb = (ir0, ir1)
    ddb = (dd0, dd1)
    rlb = (rl0, rl1)
    rrb = (rr0, rr1)
    slb = (sl0, sl1)
    srb = (sr0, sr1)

    _zero_rows(den_v, B, 128)
    _zero_rows(rl0, B, 128)
    _zero_acc(rl0, acc_num, acc_den, row0, drow0)
    pltpu.sync_copy(att_hbm, att_v)
    plsc.subcore_barrier()

    def make_idx(j, bi):
        for k in range(B // 16):
            lane = k * 16 + _iota16()
            sv = plsc.load_gather(src1d, [j * B + lane])
            plsc.store_scatter(ilb[bi], [lane], sv)
            dv = plsc.load_gather(dst1d, [j * B + lane])
            plsc.store_scatter(irb[bi], [lane], dv)
            plsc.store_scatter(ddb[bi], [lane], dv // 32)

    def issue(bi):
        pltpu.async_copy(xl_hbm.at[ilb[bi]], rlb[bi], slb[bi])
        pltpu.async_copy(xr_hbm.at[irb[bi]], rrb[bi], srb[bi])

    def wait(bi):
        pltpu.make_async_copy(xl_hbm.at[ilb[bi]], rlb[bi], slb[bi]).wait()
        pltpu.make_async_copy(xr_hbm.at[irb[bi]], rrb[bi], srb[bi]).wait()

    def compute(bi):
        rows_l = rlb[bi]
        rows_r = rrb[bi]
        for g in range(B // 16):
            rows16 = _full16(g * 16) + _iota16()
            dstg = irb[bi][pl.ds(g * 16, 16)]
            colb = (dstg - (dstg // 32) * 32) * 4

            def abody(t, a):
                cb = t * 8
                for u in range(8):
                    cid = _full16(0) + (cb + u)
                    ml = plsc.load_gather(rows_l, [rows16, cid])
                    mr = plsc.load_gather(rows_r, [rows16, cid])
                    m = ml + mr
                    m = jnp.where(m > 0, m, m * 0.2)
                    a = a + plsc.load_gather(att_v, [cid]) * m
                return a
            a = lax.fori_loop(0, 8, abody, _zeros16())
            ex = jnp.exp(a)
            ex_v[...] = ex
            plsc.store_scatter(den_v, [rows16, colb], ex)

            def mbody(ei, carry2):
                rowv = _full16(g * 16) + ei
                exb = plsc.load_gather(ex_v, [_full16(0) + ei])
                for jj in range(4):
                    cols = _full16(jj * 16) + _iota16()
                    rl = plsc.load_gather(rows_l, [rowv, cols])
                    plsc.store_scatter(rows_l, [rowv, cols], exb * rl)
                return carry2
            lax.fori_loop(0, 16, mbody, 0)
        pltpu.sync_copy(rows_l, acc_num.at[irb[bi]], add=True)
        pltpu.sync_copy(den_v, acc_den.at[ddb[bi]], add=True)
        for g in range(B // 16):
            rows16 = _full16(g * 16) + _iota16()
            dstg = irb[bi][pl.ds(g * 16, 16)]
            colb = (dstg - (dstg // 32) * 32) * 4
            plsc.store_scatter(den_v, [rows16, colb], _zeros16())

    def chunk(p, carry):
        ebase = c * (EP // 2) + s * (ITERS2 * B) + p * (SCI * B)
        pltpu.sync_copy(src_hbm.at[pl.ds(ebase, SCI * B)], src1d)
        pltpu.sync_copy(dst_hbm.at[pl.ds(ebase, SCI * B)], dst1d)
        make_idx(0, 0)
        issue(0)

        def jpair(q, carry2):
            j0 = q * 2
            wait(0)
            make_idx(j0 + 1, 1)
            issue(1)
            compute(0)
            wait(1)

            @pl.when(q < SCI // 2 - 1)
            def _prefetch():
                make_idx(j0 + 2, 0)
                issue(0)
            compute(1)
            return carry2
        lax.fori_loop(0, SCI // 2, jpair, 0)
        return carry
    lax.fori_loop(0, NCH2, chunk, 0)

    plsc.subcore_barrier()
    pltpu.sync_copy(acc_num.at[pl.ds(row0, ROWS_PT)],
                    num_hbm.at[pl.ds(c * NP + row0, ROWS_PT)])
    pltpu.sync_copy(acc_den.at[pl.ds(drow0, DROWS_PT)],
                    den_hbm.at[pl.ds(c * NPD + drow0, DROWS_PT)])


def _sc_mesh():
    return plsc.VectorSubcoreMesh(core_axis_name="c", subcore_axis_name="s")


def _edge_phase1(xl_i, xr_i, src_e, dst_e, att1f):
    f = pl.kernel(
        _edge_l1,
        out_type=[
            jax.ShapeDtypeStruct((2 * NP, 128), _f32),
            jax.ShapeDtypeStruct((2 * NPD, 128), _f32),
        ],
        mesh=_sc_mesh(),
        scratch_types=[
            pltpu.VMEM_SHARED((NP, 128), _f32),
            pltpu.VMEM_SHARED((NPD, 128), _f32),
            pltpu.VMEM((SCI * B,), _i32),
            pltpu.VMEM((SCI * B,), _i32),
            pltpu.VMEM((B,), _i32),
            pltpu.VMEM((B,), _i32),
            pltpu.VMEM((B,), _i32),
            pltpu.VMEM((B,), _i32),
            pltpu.VMEM((B,), _i32),
            pltpu.VMEM((B,), _i32),
            pltpu.VMEM((B,), _i32),
            pltpu.VMEM((B,), _i32),
            pltpu.VMEM((B, 128), _f32),
            pltpu.VMEM((B, 128), _f32),
            pltpu.VMEM((B, 128), _f32),
            pltpu.VMEM((B, 128), _f32),
            pltpu.VMEM((B, 128), _f32),
            pltpu.VMEM((64,), _f32),
            pltpu.VMEM((128,), _f32),
            pltpu.SemaphoreType.DMA,
            pltpu.SemaphoreType.DMA,
            pltpu.SemaphoreType.DMA,
            pltpu.SemaphoreType.DMA,
        ],
        compiler_params=pltpu.CompilerParams(needs_layout_passes=False),
    )
    return f(xl_i, xr_i, src_e, dst_e, att1f)


def _edge_phase2(xl2, xr2, src_e, dst_e, att2f):
    f = pl.kernel(
        _edge_l2,
        out_type=[
            jax.ShapeDtypeStruct((2 * NP, 128), _f32),
            jax.ShapeDtypeStruct((2 * NPD, 128), _f32),
        ],
        mesh=_sc_mesh(),
        scratch_types=[
            pltpu.VMEM_SHARED((NP, 128), _f32),
            pltpu.VMEM_SHARED((NPD, 128), _f32),
            pltpu.VMEM((SCI * B,), _i32),
            pltpu.VMEM((SCI * B,), _i32),
            pltpu.VMEM((B,), _i32),
            pltpu.VMEM((B,), _i32),
            pltpu.VMEM((B,), _i32),
            pltpu.VMEM((B,), _i32),
            pltpu.VMEM((B,), _i32),
            pltpu.VMEM((B,), _i32),
            pltpu.VMEM((B, 128), _f32),
            pltpu.VMEM((B, 128), _f32),
            pltpu.VMEM((B, 128), _f32),
            pltpu.VMEM((B, 128), _f32),
            pltpu.VMEM((B, 128), _f32),
            pltpu.VMEM((16,), _f32),
            pltpu.VMEM((64,), _f32),
            pltpu.SemaphoreType.DMA,
            pltpu.SemaphoreType.DMA,
            pltpu.SemaphoreType.DMA,
            pltpu.SemaphoreType.DMA,
        ],
        compiler_params=pltpu.CompilerParams(needs_layout_passes=False),
    )
    return f(xl2, xr2, src_e, dst_e, att2f)


def kernel(x, edge_index, Wl1, Wr1, att1, b1, Wl2, Wr2, att2, b2, Wlin, blin):
    x_pad = jnp.zeros((NP, 256), _f32).at[:N].set(x.astype(_f32))
    ei = edge_index.astype(_i32)
    self_i = jnp.arange(N, dtype=_i32)
    e_raw = ei.shape[1]
    pad = jnp.full((EP - e_raw - N,), DUMP, _i32)
    src_e = jnp.concatenate([ei[0], self_i, pad])
    dst_e = jnp.concatenate([ei[1], self_i, pad])

    wcat1 = jnp.concatenate([Wl1, Wr1], axis=1)
    xl_i, xr_i = _proj(x_pad, wcat1)

    att1f = att1.reshape(256).astype(_f32)
    num1, den1 = _edge_phase1(xl_i, xr_i, src_e, dst_e, att1f)
    num1 = num1.reshape(2, NP, 128)
    den1 = den1.reshape(2, NPD, 128)[:, :NPDU]
    den1 = den1.reshape(2, NP, 4)

    wcat2 = jnp.concatenate([Wl2, Wr2], axis=1)
    xl2, xr2 = _mid(num1, den1, b1.reshape(1, 256), wcat2)

    att2f = att2.reshape(64).astype(_f32)
    num2, den2 = _edge_phase2(xl2, xr2, src_e, dst_e, att2f)
    num2 = num2.reshape(2, NP, 128)
    den2 = den2.reshape(2, NPD, 128)[:, :NPDU]
    den2 = den2.reshape(2, NP, 4)

    out, prob = _head(num2, den2, b2.reshape(1, 64), Wlin,
                      blin.reshape(1, 16))
    return (out[:N], prob[:N])


# speedup vs baseline: 2.0765x; 1.3485x over previous
import functools

import jax
import jax.numpy as jnp
from jax import lax
from jax.experimental import pallas as pl
from jax.experimental.pallas import tpu as pltpu
from jax.experimental.pallas import tpu_sc as plsc

N = 10000
NP = 10240
DUMP = N
EP = 172032
B = 64
SCI = 12
ITERS1 = 168
ITERS2 = 84
NCH1 = ITERS1 // SCI
NCH2 = ITERS2 // SCI
ROWS_PT = NP // 16
NPDU = NP // 32
NPD = 512
DROWS_PT = NPD // 16
BLK = 1024

_i32 = jnp.int32
_f32 = jnp.float32


def _iota16():
    return lax.iota(_i32, 16)


def _zeros16():
    return jnp.zeros((16,), _f32)


def _full16(v):
    return jnp.full((16,), v, _i32)


def _proj_body(x_ref, w_ref, xl_ref, xr_ref):
    h = jnp.dot(x_ref[...], w_ref[...], preferred_element_type=_f32)
    blk = x_ref.shape[0]
    xl_ref[...] = h[:, :256].reshape(2 * blk, 128)
    xr_ref[...] = h[:, 256:].reshape(2 * blk, 128)


def _proj(x_pad, wcat):
    return pl.pallas_call(
        _proj_body,
        grid=(NP // BLK,),
        in_specs=[
            pl.BlockSpec((BLK, 256), lambda i: (i, 0)),
            pl.BlockSpec((256, 512), lambda i: (0, 0)),
        ],
        out_specs=[
            pl.BlockSpec((2 * BLK, 128), lambda i: (i, 0)),
            pl.BlockSpec((2 * BLK, 128), lambda i: (i, 0)),
        ],
        out_shape=[
            jax.ShapeDtypeStruct((2 * NP, 128), _f32),
            jax.ShapeDtypeStruct((2 * NP, 128), _f32),
        ],
    )(x_pad, wcat)


def _mid_body(num_ref, den_ref, b1_ref, w_ref, xl2_ref, xr2_ref):
    num = num_ref[...]
    den = den_ref[...]
    hh = lax.broadcasted_iota(_i32, (4, 128), 0)
    cc = lax.broadcasted_iota(_i32, (4, 128), 1) // 32
    rmat = jnp.where(hh == cc, 1.0, 0.0).astype(_f32)
    h0 = num[0] / (jnp.dot(den[0], rmat, preferred_element_type=_f32) + 1e-16)
    h1 = num[1] / (jnp.dot(den[1], rmat, preferred_element_type=_f32) + 1e-16)
    h = jnp.concatenate([h0, h1], axis=-1) + b1_ref[...]
    h = jnp.where(h > 0, h, 0.01 * h)
    z = jnp.dot(h, w_ref[...], preferred_element_type=_f32)
    zz = jnp.zeros_like(z[:, :64])
    xl2_ref[...] = jnp.concatenate([z[:, :64], zz], axis=-1)
    xr2_ref[...] = jnp.concatenate([z[:, 64:], zz], axis=-1)


def _mid(num1, den1, b1, wcat2):
    return pl.pallas_call(
        _mid_body,
        grid=(NP // BLK,),
        in_specs=[
            pl.BlockSpec((2, BLK, 128), lambda i: (0, i, 0)),
            pl.BlockSpec((2, BLK, 4), lambda i: (0, i, 0)),
            pl.BlockSpec((1, 256), lambda i: (0, 0)),
            pl.BlockSpec((256, 128), lambda i: (0, 0)),
        ],
        out_specs=[
            pl.BlockSpec((BLK, 128), lambda i: (i, 0)),
            pl.BlockSpec((BLK, 128), lambda i: (i, 0)),
        ],
        out_shape=[
            jax.ShapeDtypeStruct((NP, 128), _f32),
            jax.ShapeDtypeStruct((NP, 128), _f32),
        ],
    )(num1, den1, b1, wcat2)


def _head_body(num_ref, den_ref, b2_ref, wlin_ref, blin_ref, out_ref, prob_ref):
    num = num_ref[...]
    den = den_ref[...]
    d = den[0, :, 0:1] + den[1, :, 0:1]
    h2 = (num[0, :, :64] + num[1, :, :64]) / (d + 1e-16) + b2_ref[...]
    h2 = jnp.maximum(h2, 0.0)
    z = jnp.dot(h2, wlin_ref[...], preferred_element_type=_f32) + blin_ref[...]
    out_ref[...] = z
    m = jnp.max(z, axis=-1, keepdims=True)
    ez = jnp.exp(z - m)
    prob_ref[...] = ez / jnp.sum(ez, axis=-1, keepdims=True)


def _head(num2, den2, b2, wlin, blin):
    return pl.pallas_call(
        _head_body,
        grid=(NP // BLK,),
        in_specs=[
            pl.BlockSpec((2, BLK, 128), lambda i: (0, i, 0)),
            pl.BlockSpec((2, BLK, 4), lambda i: (0, i, 0)),
            pl.BlockSpec((1, 64), lambda i: (0, 0)),
            pl.BlockSpec((64, 16), lambda i: (0, 0)),
            pl.BlockSpec((1, 16), lambda i: (0, 0)),
        ],
        out_specs=[
            pl.BlockSpec((BLK, 16), lambda i: (i, 0)),
            pl.BlockSpec((BLK, 16), lambda i: (i, 0)),
        ],
        out_shape=[
            jax.ShapeDtypeStruct((NP, 16), _f32),
            jax.ShapeDtypeStruct((NP, 16), _f32),
        ],
    )(num2, den2, b2, wlin, blin)


def _zero_rows(buf, nrows, cols):
    def rzody(r, carry):
        for j in range(cols // 16):
            plsc.store_scatter(buf, [_full16(0) + r, j * 16 + _iota16()],
                               _zeros16())
        return carry
    lax.fori_loop(0, nrows, rzody, 0)


def _zero_acc(rows_l, acc_num, acc_den, row0, drow0):
    def zbody(k, carry):
        pltpu.sync_copy(rows_l, acc_num.at[pl.ds(row0 + k * B, B)])
        return carry
    lax.fori_loop(0, ROWS_PT // B, zbody, 0)
    pltpu.sync_copy(rows_l.at[pl.ds(0, DROWS_PT)],
                    acc_den.at[pl.ds(drow0, DROWS_PT)])


def _edge_l1(xl_hbm, xr_hbm, src_hbm, dst_hbm, att_hbm,
             num_hbm, den_hbm,
             acc_num, acc_den,
             src1d, dst1d,
             il0, il1, ir0, ir1, dd0, dd1, dv0, dv1,
             rl0, rl1, rows_r, m_s, den_v,
             ex_v, att_v, sl0, sl1, sem_r):
    c = lax.axis_index("c")
    s = lax.axis_index("s")
    row0 = s * ROWS_PT
    drow0 = s * DROWS_PT
    ilb = (il0, il1)
    irb = (ir0, ir1)
    ddb = (dd0, dd1)
    dvb = (dv0, dv1)
    rlb = (rl0, rl1)
    slb = (sl0, sl1)

    _zero_rows(den_v, B, 128)
    _zero_rows(rl0, B, 128)
    _zero_acc(rl0, acc_num, acc_den, row0, drow0)
    pltpu.sync_copy(att_hbm.at[pl.ds(c * 128, 128)], att_v)
    plsc.subcore_barrier()

    def make_idx(j, bi):
        for k in range(B // 16):
            lane = k * 16 + _iota16()
            sv = plsc.load_gather(src1d, [j * B + lane])
            plsc.store_scatter(ilb[bi], [lane], sv * 2 + c)
            dv = plsc.load_gather(dst1d, [j * B + lane])
            plsc.store_scatter(irb[bi], [lane], dv * 2 + c)
            plsc.store_scatter(ddb[bi], [lane], dv // 32)
            plsc.store_scatter(dvb[bi], [lane], dv)

    def issue(bi):
        pltpu.async_copy(xl_hbm.at[ilb[bi]], rlb[bi], slb[bi])
        pltpu.async_copy(xr_hbm.at[irb[bi]], rows_r, sem_r)

    def wait(bi):
        pltpu.make_async_copy(xl_hbm.at[ilb[bi]], rlb[bi], slb[bi]).wait()
        pltpu.make_async_copy(xr_hbm.at[irb[bi]], rows_r, sem_r).wait()

    def build(bi):
        rows_l = rlb[bi]

        def bbody(e, carry2):
            for k in range(8):
                lane = k * 16 + _iota16()
                ml = plsc.load_gather(rows_l, [_full16(0) + e, lane])
                mr = plsc.load_gather(rows_r, [_full16(0) + e, lane])
                m = ml + mr
                m = jnp.where(m > 0, m, m * 0.2)
                plsc.store_scatter(m_s, [e * 129 + lane], m)
            return carry2
        lax.fori_loop(0, B, bbody, 0)

    def compute(bi):
        rows_l = rlb[bi]
        for g in range(B // 16):
            rows16 = _full16(g * 16) + _iota16()
            rbase = rows16 * 129
            dstg = dvb[bi][pl.ds(g * 16, 16)]
            colb = (dstg - (dstg // 32) * 32) * 4
            for h in range(4):
                def abody(t, a):
                    cb = h * 32 + t * 8
                    for u in range(8):
                        cid = _full16(0) + (cb + u)
                        mv = plsc.load_gather(m_s, [rbase + (cb + u)])
                        a = a + plsc.load_gather(att_v, [cid]) * mv
                    return a
                a = lax.fori_loop(0, 4, abody, _zeros16())
                ex = jnp.exp(a)
                ex_v[pl.ds(h * 16, 16)] = ex
                plsc.store_scatter(den_v, [rows16, colb + h], ex)

            def mbody(ei, carry2):
                rowv = _full16(g * 16) + ei
                for h in range(4):
                    exb = plsc.load_gather(ex_v, [_full16(h * 16) + ei])
                    for jj in range(2):
                        cols = _full16(h * 32 + jj * 16) + _iota16()
                        rl = plsc.load_gather(rows_l, [rowv, cols])
                        plsc.store_scatter(rows_l, [rowv, cols], exb * rl)
                return carry2
            lax.fori_loop(0, 16, mbody, 0)
        pltpu.sync_copy(rows_l, acc_num.at[dvb[bi]], add=True)
        pltpu.sync_copy(den_v, acc_den.at[ddb[bi]], add=True)
        for g in range(B // 16):
            rows16 = _full16(g * 16) + _iota16()
            dstg = dvb[bi][pl.ds(g * 16, 16)]
            colb = (dstg - (dstg // 32) * 32) * 4
            for h in range(4):
                plsc.store_scatter(den_v, [rows16, colb + h], _zeros16())

    def chunk(p, carry):
        ebase = s * (ITERS1 * B) + p * (SCI * B)
        pltpu.sync_copy(src_hbm.at[pl.ds(ebase, SCI * B)], src1d)
        pltpu.sync_copy(dst_hbm.at[pl.ds(ebase, SCI * B)], dst1d)
        make_idx(0, 0)
        issue(0)

        def jpair(q, carry2):
            j0 = q * 2
            wait(0)
            build(0)
            make_idx(j0 + 1, 1)
            issue(1)
            compute(0)
            wait(1)
            build(1)

            @pl.when(q < SCI // 2 - 1)
            def _prefetch():
                make_idx(j0 + 2, 0)
                issue(0)
            compute(1)
            return carry2
        lax.fori_loop(0, SCI // 2, jpair, 0)
        return carry
    lax.fori_loop(0, NCH1, chunk, 0)

    plsc.subcore_barrier()
    pltpu.sync_copy(acc_num.at[pl.ds(row0, ROWS_PT)],
                    num_hbm.at[pl.ds(c * NP + row0, ROWS_PT)])
    pltpu.sync_copy(acc_den.at[pl.ds(drow0, DROWS_PT)],
                    den_hbm.at[pl.ds(c * NPD + drow0, DROWS_PT)])


def _edge_l2(xl_hbm, xr_hbm, src_hbm, dst_hbm, att_hbm,
             num_hbm, den_hbm,
             acc_num, acc_den,
             src1d, dst1d,
             il0, il1, ir0, ir1, dd0, dd1,
             rl0, rl1, rows_r, m_s, den_v,
             ex_v, att_v, sl0, sl1, sem_r):
    c = lax.axis_index("c")
    s = lax.axis_index("s")
    row0 = s * ROWS_PT
    drow0 = s * DROWS_PT
    ilb = (il0, il1)
    irb = (ir0, ir1)
    ddb = (dd0, dd1)
    rlb = (rl0, rl1)
    slb = (sl0, sl1)

    _zero_rows(den_v, B, 128)
    _zero_rows(rl0, B, 128)
    _zero_acc(rl0, acc_num, acc_den, row0, drow0)
    pltpu.sync_copy(att_hbm, att_v)
    plsc.subcore_barrier()

    def make_idx(j, bi):
        for k in range(B // 16):
            lane = k * 16 + _iota16()
            sv = plsc.load_gather(src1d, [j * B + lane])
            plsc.store_scatter(ilb[bi], [lane], sv)
            dv = plsc.load_gather(dst1d, [j * B + lane])
            plsc.store_scatter(irb[bi], [lane], dv)
            plsc.store_scatter(ddb[bi], [lane], dv // 32)

    def issue(bi):
        pltpu.async_copy(xl_hbm.at[ilb[bi]], rlb[bi], slb[bi])
        pltpu.async_copy(xr_hbm.at[irb[bi]], rows_r, sem_r)

    def wait(bi):
        pltpu.make_async_copy(xl_hbm.at[ilb[bi]], rlb[bi], slb[bi]).wait()
        pltpu.make_async_copy(xr_hbm.at[irb[bi]], rows_r, sem_r).wait()

    def build(bi):
        rows_l = rlb[bi]

        def bbody(e, carry2):
            for k in range(4):
                lane = k * 16 + _iota16()
                ml = plsc.load_gather(rows_l, [_full16(0) + e, lane])
                mr = plsc.load_gather(rows_r, [_full16(0) + e, lane])
                m = ml + mr
                m = jnp.where(m > 0, m, m * 0.2)
                plsc.store_scatter(m_s, [e * 65 + lane], m)
            return carry2
        lax.fori_loop(0, B, bbody, 0)

    def compute(bi):
        rows_l = rlb[bi]
        for g in range(B // 16):
            rows16 = _full16(g * 16) + _iota16()
            rbase = rows16 * 65
            dstg = irb[bi][pl.ds(g * 16, 16)]
            colb = (dstg - (dstg // 32) * 32) * 4

            def abody(t, a):
                cb = t * 8
                for u in range(8):
                    cid = _full16(0) + (cb + u)
                    mv = plsc.load_gather(m_s, [rbase + (cb + u)])
                    a = a + plsc.load_gather(att_v, [cid]) * mv
                return a
            a = lax.fori_loop(0, 8, abody, _zeros16())
            ex = jnp.exp(a)
            ex_v[...] = ex
            plsc.store_scatter(den_v, [rows16, colb], ex)

            def mbody(ei, carry2):
                rowv = _full16(g * 16) + ei
                exb = plsc.load_gather(ex_v, [_full16(0) + ei])
                for jj in range(4):
                    cols = _full16(jj * 16) + _iota16()
                    rl = plsc.load_gather(rows_l, [rowv, cols])
                    plsc.store_scatter(rows_l, [rowv, cols], exb * rl)
                return carry2
            lax.fori_loop(0, 16, mbody, 0)
        pltpu.sync_copy(rows_l, acc_num.at[irb[bi]], add=True)
        pltpu.sync_copy(den_v, acc_den.at[ddb[bi]], add=True)
        for g in range(B // 16):
            rows16 = _full16(g * 16) + _iota16()
            dstg = irb[bi][pl.ds(g * 16, 16)]
            colb = (dstg - (dstg // 32) * 32) * 4
            plsc.store_scatter(den_v, [rows16, colb], _zeros16())

    def chunk(p, carry):
        ebase = c * (EP // 2) + s * (ITERS2 * B) + p * (SCI * B)
        pltpu.sync_copy(src_hbm.at[pl.ds(ebase, SCI * B)], src1d)
        pltpu.sync_copy(dst_hbm.at[pl.ds(ebase, SCI * B)], dst1d)
        make_idx(0, 0)
        issue(0)

        def jpair(q, carry2):
            j0 = q * 2
            wait(0)
            build(0)
            make_idx(j0 + 1, 1)
            issue(1)
            compute(0)
            wait(1)
            build(1)

            @pl.when(q < SCI // 2 - 1)
            def _prefetch():
                make_idx(j0 + 2, 0)
                issue(0)
            compute(1)
            return carry2
        lax.fori_loop(0, SCI // 2, jpair, 0)
        return carry
    lax.fori_loop(0, NCH2, chunk, 0)

    plsc.subcore_barrier()
    pltpu.sync_copy(acc_num.at[pl.ds(row0, ROWS_PT)],
                    num_hbm.at[pl.ds(c * NP + row0, ROWS_PT)])
    pltpu.sync_copy(acc_den.at[pl.ds(drow0, DROWS_PT)],
                    den_hbm.at[pl.ds(c * NPD + drow0, DROWS_PT)])


def _sc_mesh():
    return plsc.VectorSubcoreMesh(core_axis_name="c", subcore_axis_name="s")


def _edge_phase1(xl_i, xr_i, src_e, dst_e, att1f):
    f = pl.kernel(
        _edge_l1,
        out_type=[
            jax.ShapeDtypeStruct((2 * NP, 128), _f32),
            jax.ShapeDtypeStruct((2 * NPD, 128), _f32),
        ],
        mesh=_sc_mesh(),
        scratch_types=[
            pltpu.VMEM_SHARED((NP, 128), _f32),
            pltpu.VMEM_SHARED((NPD, 128), _f32),
            pltpu.VMEM((SCI * B,), _i32),
            pltpu.VMEM((SCI * B,), _i32),
            pltpu.VMEM((B,), _i32),
            pltpu.VMEM((B,), _i32),
            pltpu.VMEM((B,), _i32),
            pltpu.VMEM((B,), _i32),
            pltpu.VMEM((B,), _i32),
            pltpu.VMEM((B,), _i32),
            pltpu.VMEM((B,), _i32),
            pltpu.VMEM((B,), _i32),
            pltpu.VMEM((B, 128), _f32),
            pltpu.VMEM((B, 128), _f32),
            pltpu.VMEM((B, 128), _f32),
            pltpu.VMEM((B * 129,), _f32),
            pltpu.VMEM((B, 128), _f32),
            pltpu.VMEM((64,), _f32),
            pltpu.VMEM((128,), _f32),
            pltpu.SemaphoreType.DMA,
            pltpu.SemaphoreType.DMA,
            pltpu.SemaphoreType.DMA,
        ],
        compiler_params=pltpu.CompilerParams(needs_layout_passes=False),
    )
    return f(xl_i, xr_i, src_e, dst_e, att1f)


def _edge_phase2(xl2, xr2, src_e, dst_e, att2f):
    f = pl.kernel(
        _edge_l2,
        out_type=[
            jax.ShapeDtypeStruct((2 * NP, 128), _f32),
            jax.ShapeDtypeStruct((2 * NPD, 128), _f32),
        ],
        mesh=_sc_mesh(),
        scratch_types=[
            pltpu.VMEM_SHARED((NP, 128), _f32),
            pltpu.VMEM_SHARED((NPD, 128), _f32),
            pltpu.VMEM((SCI * B,), _i32),
            pltpu.VMEM((SCI * B,), _i32),
            pltpu.VMEM((B,), _i32),
            pltpu.VMEM((B,), _i32),
            pltpu.VMEM((B,), _i32),
            pltpu.VMEM((B,), _i32),
            pltpu.VMEM((B,), _i32),
            pltpu.VMEM((B,), _i32),
            pltpu.VMEM((B, 128), _f32),
            pltpu.VMEM((B, 128), _f32),
            pltpu.VMEM((B, 128), _f32),
            pltpu.VMEM((B * 65,), _f32),
            pltpu.VMEM((B, 128), _f32),
            pltpu.VMEM((16,), _f32),
            pltpu.VMEM((64,), _f32),
            pltpu.SemaphoreType.DMA,
            pltpu.SemaphoreType.DMA,
            pltpu.SemaphoreType.DMA,
        ],
        compiler_params=pltpu.CompilerParams(needs_layout_passes=False),
    )
    return f(xl2, xr2, src_e, dst_e, att2f)


def kernel(x, edge_index, Wl1, Wr1, att1, b1, Wl2, Wr2, att2, b2, Wlin, blin):
    x_pad = jnp.zeros((NP, 256), _f32).at[:N].set(x.astype(_f32))
    ei = edge_index.astype(_i32)
    self_i = jnp.arange(N, dtype=_i32)
    e_raw = ei.shape[1]
    pad = jnp.full((EP - e_raw - N,), DUMP, _i32)
    src_e = jnp.concatenate([ei[0], self_i, pad])
    dst_e = jnp.concatenate([ei[1], self_i, pad])

    wcat1 = jnp.concatenate([Wl1, Wr1], axis=1)
    xl_i, xr_i = _proj(x_pad, wcat1)

    att1f = att1.reshape(256).astype(_f32)
    num1, den1 = _edge_phase1(xl_i, xr_i, src_e, dst_e, att1f)
    num1 = num1.reshape(2, NP, 128)
    den1 = den1.reshape(2, NPD, 128)[:, :NPDU]
    den1 = den1.reshape(2, NP, 4)

    wcat2 = jnp.concatenate([Wl2, Wr2], axis=1)
    xl2, xr2 = _mid(num1, den1, b1.reshape(1, 256), wcat2)

    att2f = att2.reshape(64).astype(_f32)
    num2, den2 = _edge_phase2(xl2, xr2, src_e, dst_e, att2f)
    num2 = num2.reshape(2, NP, 128)
    den2 = den2.reshape(2, NPD, 128)[:, :NPDU]
    den2 = den2.reshape(2, NP, 4)

    out, prob = _head(num2, den2, b2.reshape(1, 64), Wlin,
                      blin.reshape(1, 16))
    return (out[:N], prob[:N])
